# baseline (speedup 1.0000x reference)
import functools

import jax
import jax.numpy as jnp
from jax import lax
from jax.experimental import pallas as pl
from jax.experimental.pallas import tpu as pltpu
from jax.experimental.pallas import tpu_sc as plsc

B = 4096
D = 64
NW = 32
BPW = B // NW


def _gather_body(user_hbm, pos_hbm, neg_hbm, user_emb_hbm, item_emb_hbm,
                 u_out, i_out, j_out, idx_v, rows_v, sem):
  wid = lax.axis_index("s") * 2 + lax.axis_index("c")
  base = wid * BPW
  for idx_hbm, table_hbm, out_hbm in (
      (user_hbm, user_emb_hbm, u_out),
      (pos_hbm, item_emb_hbm, i_out),
      (neg_hbm, item_emb_hbm, j_out),
  ):
    pltpu.sync_copy(idx_hbm.at[pl.ds(base, BPW)], idx_v)
    pltpu.async_copy(table_hbm.at[idx_v], rows_v, sem).wait()
    pltpu.sync_copy(rows_v, out_hbm.at[pl.ds(base, BPW)])


_sc_gather = functools.partial(
    pl.kernel,
    out_type=[jax.ShapeDtypeStruct((B, D), jnp.float32)] * 3,
    mesh=plsc.VectorSubcoreMesh(core_axis_name="c", subcore_axis_name="s"),
    scratch_types=[
        pltpu.VMEM((BPW,), jnp.int32),
        pltpu.VMEM((BPW, D), jnp.float32),
        pltpu.SemaphoreType.DMA,
    ],
    compiler_params=pltpu.CompilerParams(use_tc_tiling_on_sc=False),
)(_gather_body)


BM = 512


def _score_body(u_ref, ii_ref, ij_ref, oi_ref, oj_ref):
  u = u_ref[...]
  dn = (((1,), (1,)), ((), ()))
  oi_ref[...] = lax.dot_general(u, ii_ref[...], dn,
                                preferred_element_type=jnp.float32)
  oj_ref[...] = lax.dot_general(u, ij_ref[...], dn,
                                preferred_element_type=jnp.float32)


def _scores(u, item_i, item_j):
  return pl.pallas_call(
      _score_body,
      grid=(B // BM,),
      in_specs=[
          pl.BlockSpec((BM, D), lambda i: (i, 0)),
          pl.BlockSpec((B, D), lambda i: (0, 0)),
          pl.BlockSpec((B, D), lambda i: (0, 0)),
      ],
      out_specs=[
          pl.BlockSpec((BM, B), lambda i: (i, 0)),
          pl.BlockSpec((BM, B), lambda i: (i, 0)),
      ],
      out_shape=[jax.ShapeDtypeStruct((B, B), jnp.float32)] * 2,
  )(u, item_i, item_j)


@jax.jit
def kernel(user, pos_item, neg_item, user_emb, item_emb):
  u, item_i, item_j = _sc_gather(user, pos_item, neg_item, user_emb, item_emb)
  return tuple(_scores(u, item_i, item_j))


# speedup vs baseline: 1.1353x; 1.1353x over previous
import functools

import jax
import jax.numpy as jnp
from jax import lax
from jax.experimental import pallas as pl
from jax.experimental.pallas import tpu as pltpu
from jax.experimental.pallas import tpu_sc as plsc

B = 4096
D = 64
NW = 32
BPW = B // NW


def _gather_body(user_hbm, pos_hbm, neg_hbm, user_emb_hbm, item_emb_hbm,
                 u_out, i_out, j_out, idx_v, sem):
  wid = lax.axis_index("s") * 2 + lax.axis_index("c")
  base = wid * BPW
  for idx_hbm, table_hbm, out_hbm in (
      (user_hbm, user_emb_hbm, u_out),
      (pos_hbm, item_emb_hbm, i_out),
      (neg_hbm, item_emb_hbm, j_out),
  ):
    pltpu.sync_copy(idx_hbm.at[pl.ds(base, BPW)], idx_v)
    descs = []
    for c in range(BPW // 16):
      vec = idx_v[pl.ds(c * 16, 16)]
      for r in range(16):
        d = pltpu.make_async_copy(
            table_hbm.at[pl.ds(vec[r], 1)],
            out_hbm.at[pl.ds(base + c * 16 + r, 1)],
            sem,
        )
        d.start()
        descs.append(d)
    for d in descs:
      d.wait()


_sc_gather = functools.partial(
    pl.kernel,
    out_type=[jax.ShapeDtypeStruct((B, D), jnp.float32)] * 3,
    mesh=plsc.VectorSubcoreMesh(core_axis_name="c", subcore_axis_name="s"),
    scratch_types=[
        pltpu.VMEM((BPW,), jnp.int32),
        pltpu.SemaphoreType.DMA,
    ],
)(_gather_body)


BM = 512


def _score_body(u_ref, ii_ref, ij_ref, oi_ref, oj_ref):
  u = u_ref[...]
  dn = (((1,), (1,)), ((), ()))
  oi_ref[...] = lax.dot_general(u, ii_ref[...], dn,
                                preferred_element_type=jnp.float32)
  oj_ref[...] = lax.dot_general(u, ij_ref[...], dn,
                                preferred_element_type=jnp.float32)


def _scores(u, item_i, item_j):
  return pl.pallas_call(
      _score_body,
      grid=(B // BM,),
      in_specs=[
          pl.BlockSpec((BM, D), lambda i: (i, 0)),
          pl.BlockSpec((B, D), lambda i: (0, 0)),
          pl.BlockSpec((B, D), lambda i: (0, 0)),
      ],
      out_specs=[
          pl.BlockSpec((BM, B), lambda i: (i, 0)),
          pl.BlockSpec((BM, B), lambda i: (i, 0)),
      ],
      out_shape=[jax.ShapeDtypeStruct((B, B), jnp.float32)] * 2,
  )(u, item_i, item_j)


@jax.jit
def kernel(user, pos_item, neg_item, user_emb, item_emb):
  u, item_i, item_j = _sc_gather(user, pos_item, neg_item, user_emb, item_emb)
  return tuple(_scores(u, item_i, item_j))


# speedup vs baseline: 2.7993x; 2.4658x over previous
import functools

import jax
import jax.numpy as jnp
from jax import lax
from jax.experimental import pallas as pl
from jax.experimental.pallas import tpu as pltpu
from jax.experimental.pallas import tpu_sc as plsc

B = 4096
D = 64
NW = 32
BPW = B // NW
NBUF = 8


def _gather_one(idx_hbm, table_hbm, out_hbm, base, idx_v, blk_v, rows_v, sems):
  pltpu.sync_copy(idx_hbm.at[pl.ds(base, BPW)], idx_v)

  def chunk(c, carry):
    vec = idx_v[pl.ds(c * 16, 16)]
    for w in range(16 // NBUF):
      descs = []
      for b in range(NBUF):
        idx = vec[w * NBUF + b]
        loff = pl.multiple_of((idx >> 7) * 128, 128)
        d = pltpu.make_async_copy(
            table_hbm.at[:, pl.ds(loff, 128)],
            blk_v.at[b],
            sems.at[b],
        )
        d.start()
        descs.append(d)
      for b in range(NBUF):
        descs[b].wait()
        idx = vec[w * NBUF + b]
        m = jnp.broadcast_to(idx & 127, (16,))
        k = jnp.broadcast_to(c * 16 + w * NBUF + b, (16,))
        for s in range(D // 16):
          rows = lax.broadcasted_iota(jnp.int32, (16,), 0) + (16 * s)
          col = plsc.load_gather(blk_v.at[b], [rows, m])
          plsc.store_scatter(rows_v, [rows, k], col)
    return carry

  lax.fori_loop(0, BPW // 16, chunk, 0)
  pltpu.sync_copy(rows_v, out_hbm.at[:, pl.ds(pl.multiple_of(base, 128), BPW)])


def _gather_body(user_hbm, pos_hbm, neg_hbm, uet_hbm, iet_hbm,
                 u_out, i_out, j_out, idx_v, blk_v, rows_v, sems):
  wid = lax.axis_index("s") * 2 + lax.axis_index("c")
  base = wid * BPW
  for idx_hbm, table_hbm, out_hbm in (
      (user_hbm, uet_hbm, u_out),
      (pos_hbm, iet_hbm, i_out),
      (neg_hbm, iet_hbm, j_out),
  ):
    _gather_one(idx_hbm, table_hbm, out_hbm, base, idx_v, blk_v, rows_v, sems)


_sc_gather = functools.partial(
    pl.kernel,
    out_type=[jax.ShapeDtypeStruct((D, B), jnp.float32)] * 3,
    mesh=plsc.VectorSubcoreMesh(core_axis_name="c", subcore_axis_name="s"),
    scratch_types=[
        pltpu.VMEM((BPW,), jnp.int32),
        pltpu.VMEM((NBUF, D, 128), jnp.float32),
        pltpu.VMEM((D, BPW), jnp.float32),
        pltpu.SemaphoreType.DMA((NBUF,)),
    ],
    compiler_params=pltpu.CompilerParams(disable_bounds_checks=True, needs_layout_passes=False),
)(_gather_body)


BM = 512


def _score_body(u_ref, ii_ref, ij_ref, oi_ref, oj_ref):
  u = u_ref[...]
  dn = (((0,), (0,)), ((), ()))
  oi_ref[...] = lax.dot_general(u, ii_ref[...], dn,
                                preferred_element_type=jnp.float32)
  oj_ref[...] = lax.dot_general(u, ij_ref[...], dn,
                                preferred_element_type=jnp.float32)


def _scores(u_t, item_i_t, item_j_t):
  return pl.pallas_call(
      _score_body,
      grid=(B // BM,),
      in_specs=[
          pl.BlockSpec((D, BM), lambda i: (0, i)),
          pl.BlockSpec((D, B), lambda i: (0, 0)),
          pl.BlockSpec((D, B), lambda i: (0, 0)),
      ],
      out_specs=[
          pl.BlockSpec((BM, B), lambda i: (i, 0)),
          pl.BlockSpec((BM, B), lambda i: (i, 0)),
      ],
      out_shape=[jax.ShapeDtypeStruct((B, B), jnp.float32)] * 2,
  )(u_t, item_i_t, item_j_t)


@jax.jit
def kernel(user, pos_item, neg_item, user_emb, item_emb):
  u_t, item_i_t, item_j_t = _sc_gather(
      user, pos_item, neg_item, user_emb.T, item_emb.T)
  return tuple(_scores(u_t, item_i_t, item_j_t))
